# baseline (speedup 1.0000x reference)
import functools

import numpy as np
import jax
import jax.numpy as jnp
from jax import lax
from jax.experimental import pallas as pl
from jax.experimental.pallas import tpu as pltpu
from jax.experimental.pallas import tpu_sc as plsc

BATCH = 32
SEQ = 2048
D = 128
SCALE = float(np.sqrt(128.0))

_info = plsc.get_sparse_core_info()
NC, NS, L = _info.num_cores, _info.num_subcores, _info.num_lanes
NW = NC * NS
BAND = SEQ // NW
CH_B = 4
N_CHUNK = BATCH // CH_B
CH_ROWS = CH_B * BAND
NBUF = 3


def _pos_encoding() -> np.ndarray:
    positions = np.arange(SEQ)[:, np.newaxis].astype(np.float64)
    depths = np.arange(D)[np.newaxis, :] / D
    angle_rates = 1.0 / (10000.0 ** depths)
    pe = positions * angle_rates
    pe[:, 1::2] = np.cos(pe[:, 1::2])
    pe[:, 0::2] = np.sin(pe[:, 0::2])
    return pe.astype(np.float32)


_POS = _pos_encoding()

_mesh = plsc.VectorSubcoreMesh(core_axis_name="c", subcore_axis_name="s")


@functools.partial(
    pl.kernel,
    mesh=_mesh,
    out_type=jax.ShapeDtypeStruct((BATCH, SEQ, D), jnp.float32),
    scratch_types=[
        pltpu.VMEM((BATCH, 2 * BAND), jnp.int32),
        pltpu.VMEM((BAND,), jnp.int32),
        pltpu.VMEM((BAND, D), jnp.float32),
        pltpu.VMEM((CH_ROWS, D), jnp.float32),
        pltpu.VMEM((CH_ROWS, D), jnp.float32),
        pltpu.VMEM((CH_ROWS, D), jnp.float32),
        pltpu.SemaphoreType.DMA,
        pltpu.SemaphoreType.DMA,
        pltpu.SemaphoreType.DMA,
        pltpu.SemaphoreType.DMA,
        pltpu.SemaphoreType.DMA,
        pltpu.SemaphoreType.DMA,
        pltpu.SemaphoreType.DMA,
        pltpu.SemaphoreType.DMA,
    ],
)
def _emb_kernel(x_hbm, pos_hbm, table_hbm, out_hbm,
                idx_v, pidx_v, pos_v, rows0, rows1, rows2,
                sem_i, sem_p, sem_g0, sem_g1, sem_g2, sem_o0, sem_o1, sem_o2):
    wid = lax.axis_index("s") * NC + lax.axis_index("c")
    s0 = wid * BAND
    off = (wid % 2) * BAND
    win = pl.multiple_of((wid // 2) * (2 * BAND), 2 * BAND)
    rows = (rows0, rows1, rows2)
    sem_g = (sem_g0, sem_g1, sem_g2)
    sem_o = (sem_o0, sem_o1, sem_o2)

    cp_i = pltpu.async_copy(x_hbm.at[:, pl.ds(win, 2 * BAND)], idx_v, sem_i)
    for j in range(BAND // L):
        pidx_v[pl.ds(j * L, L)] = lax.iota(jnp.int32, L) + (s0 + j * L)
    cp_p = pltpu.async_copy(pos_hbm.at[pidx_v], pos_v, sem_p)
    cp_i.wait()

    def start_chunk(c, slot):
        return [
            pltpu.async_copy(
                table_hbm.at[idx_v.at[c * CH_B + bb, pl.ds(off, BAND)]],
                rows[slot].at[pl.ds(bb * BAND, BAND)], sem_g[slot])
            for bb in range(CH_B)
        ]

    gath = [None, None, None]
    outc = [[], [], []]
    gath[0] = start_chunk(0, 0)
    gath[1] = start_chunk(1, 1)
    cp_p.wait()

    for c in range(N_CHUNK):
        s = c % NBUF
        if c + 2 < N_CHUNK:
            s2 = (c + 2) % NBUF
            for h in outc[s2]:
                h.wait()
            outc[s2] = []
            gath[s2] = start_chunk(c + 2, s2)
        for h in gath[s]:
            h.wait()
        buf = rows[s]

        def body_i(i, carry, buf=buf):
            pv = [pos_v[i, pl.ds(j * L, L)] for j in range(D // L)]
            for bb in range(CH_B):
                r = bb * BAND + i
                for j in range(D // L):
                    sl = pl.ds(j * L, L)
                    buf[r, sl] = buf[r, sl] * SCALE + pv[j]
            return carry

        lax.fori_loop(0, BAND, body_i, 0)

        for bb in range(CH_B):
            outc[s].append(pltpu.async_copy(
                buf.at[pl.ds(bb * BAND, BAND)],
                out_hbm.at[c * CH_B + bb, pl.ds(s0, BAND), :],
                sem_o[s]))
    for hs in outc:
        for h in hs:
            h.wait()


def kernel(x, table):
    return _emb_kernel(x, jnp.asarray(_POS), table)
